# baseline (speedup 1.0000x reference)
import jax
import jax.numpy as jnp
from jax import lax
from jax.experimental import pallas as pl
from jax.experimental.pallas import tpu as pltpu
from jax.experimental.pallas import tpu_sc as plsc

_B, _C, _N = 16, 256, 16384
_K = 4
_TOTAL_IN = _B * _C * _N
_TOTAL_OUT = _TOTAL_IN // _K
_NC, _NS = 2, 16
_NW = _NC * _NS
_IN_PER_W = _TOTAL_IN // _NW
_CH_IN = 32768
_CH_OUT = _CH_IN // _K
_N_CHUNKS = _IN_PER_W // _CH_IN
_GROUPS = _CH_IN // 64


def _pool_body(x_hbm, out_hbm, x_v, out_v):
    wid = lax.axis_index("s") * _NC + lax.axis_index("c")
    in_base = wid * _IN_PER_W
    out_base = wid * (_IN_PER_W // _K)
    lane = lax.broadcasted_iota(jnp.int32, (16,), 0)
    idx0 = lane * _K

    def chunk_body(c, carry):
        pltpu.sync_copy(x_hbm.at[pl.ds(in_base + c * _CH_IN, _CH_IN)], x_v)

        def group_body(i, inner):
            i0 = idx0 + i * 64
            a = plsc.load_gather(x_v, [i0])
            b = plsc.load_gather(x_v, [i0 + 1])
            c2 = plsc.load_gather(x_v, [i0 + 2])
            d = plsc.load_gather(x_v, [i0 + 3])
            out_v[pl.ds(i * 16, 16)] = (a + b + c2 + d) * 0.25
            return inner

        lax.fori_loop(0, _GROUPS, group_body, 0)
        pltpu.sync_copy(out_v, out_hbm.at[pl.ds(out_base + c * _CH_OUT, _CH_OUT)])
        return carry

    lax.fori_loop(0, _N_CHUNKS, chunk_body, 0)


def kernel(x):
    xf = x.reshape(_TOTAL_IN)
    mesh = plsc.VectorSubcoreMesh(core_axis_name="c", subcore_axis_name="s")
    out = pl.kernel(
        _pool_body,
        out_type=jax.ShapeDtypeStruct((_TOTAL_OUT,), jnp.float32),
        mesh=mesh,
        scratch_types=[
            pltpu.VMEM((_CH_IN,), jnp.float32),
            pltpu.VMEM((_CH_OUT,), jnp.float32),
        ],
        compiler_params=pltpu.CompilerParams(needs_layout_passes=False),
    )(xf)
    return out.reshape(_B, _C, _N // _K)


# speedup vs baseline: 1.8194x; 1.8194x over previous
import jax
import jax.numpy as jnp
from jax import lax
from jax.experimental import pallas as pl
from jax.experimental.pallas import tpu as pltpu
from jax.experimental.pallas import tpu_sc as plsc

_B, _C, _N = 16, 256, 16384
_K = 4
_TOTAL_IN = _B * _C * _N
_TOTAL_OUT = _TOTAL_IN // _K
_NC, _NS = 2, 16
_NW = _NC * _NS
_IN_PER_W = _TOTAL_IN // _NW
_CH_IN = 32768
_CH_OUT = _CH_IN // _K
_N_CHUNKS = _IN_PER_W // _CH_IN
_GROUPS = _CH_IN // 64


def _pool_body(x_hbm, out_hbm, xv0, xv1, ov0, ov1, is0, is1, os0, os1):
    wid = lax.axis_index("s") * _NC + lax.axis_index("c")
    in_base = wid * _IN_PER_W
    out_base = wid * (_IN_PER_W // _K)
    lane = lax.broadcasted_iota(jnp.int32, (16,), 0)
    idx = [lane * _K + r for r in range(_K)]
    xvs, ovs = (xv0, xv1), (ov0, ov1)
    isems, osems = (is0, is1), (os0, os1)

    pltpu.async_copy(x_hbm.at[pl.ds(in_base, _CH_IN)], xv0, is0)

    def pair_body(c0, carry):
        for b in (0, 1):
            c = 2 * c0 + b
            pltpu.make_async_copy(
                x_hbm.at[pl.ds(0, _CH_IN)], xvs[b], isems[b]).wait()

            @pl.when(c + 1 < _N_CHUNKS)
            def _():
                pltpu.async_copy(
                    x_hbm.at[pl.ds(in_base + (c + 1) * _CH_IN, _CH_IN)],
                    xvs[1 - b], isems[1 - b])

            @pl.when(c0 >= 1)
            def _():
                pltpu.make_async_copy(
                    ovs[b], out_hbm.at[pl.ds(0, _CH_OUT)], osems[b]).wait()

            x_v, out_v = xvs[b], ovs[b]

            @plsc.parallel_loop(0, _GROUPS, unroll=4)
            def _(i):
                sl = x_v.at[pl.ds(i * 64, 64)]
                a = plsc.load_gather(sl, [idx[0]])
                bb = plsc.load_gather(sl, [idx[1]])
                cc = plsc.load_gather(sl, [idx[2]])
                dd = plsc.load_gather(sl, [idx[3]])
                out_v[pl.ds(i * 16, 16)] = (a + bb + cc + dd) * 0.25

            pltpu.async_copy(
                out_v, out_hbm.at[pl.ds(out_base + c * _CH_OUT, _CH_OUT)],
                osems[b])
        return carry

    lax.fori_loop(0, _N_CHUNKS // 2, pair_body, 0)
    for b in (0, 1):
        pltpu.make_async_copy(
            ovs[b], out_hbm.at[pl.ds(0, _CH_OUT)], osems[b]).wait()


def kernel(x):
    xf = x.reshape(_TOTAL_IN)
    mesh = plsc.VectorSubcoreMesh(core_axis_name="c", subcore_axis_name="s")
    out = pl.kernel(
        _pool_body,
        out_type=jax.ShapeDtypeStruct((_TOTAL_OUT,), jnp.float32),
        mesh=mesh,
        scratch_types=[
            pltpu.VMEM((_CH_IN,), jnp.float32),
            pltpu.VMEM((_CH_IN,), jnp.float32),
            pltpu.VMEM((_CH_OUT,), jnp.float32),
            pltpu.VMEM((_CH_OUT,), jnp.float32),
            pltpu.SemaphoreType.DMA,
            pltpu.SemaphoreType.DMA,
            pltpu.SemaphoreType.DMA,
            pltpu.SemaphoreType.DMA,
        ],
        compiler_params=pltpu.CompilerParams(needs_layout_passes=False),
    )(xf)
    return out.reshape(_B, _C, _N // _K)
